# baseline (speedup 1.0000x reference)
import functools

import jax
import jax.numpy as jnp
from jax import lax
from jax.experimental import pallas as pl
from jax.experimental.pallas import tpu as pltpu
from jax.experimental.pallas import tpu_sc as plsc

N = 10000
D = 128
E = 320000

NC = 2
NS = 16
NW = NC * NS
CHUNK = 128
NCHUNKS = 80
E_PER_TILE = CHUNK * NCHUNKS
E_PAD = E_PER_TILE * NW
N_SLICE = 632
N_PAD = N_SLICE * NS
ROWS_TC = 400
GRID_TC = N // ROWS_TC


C_CORE0 = 136
C_CORE1 = 2 * NCHUNKS - C_CORE0
IDXW = 8


def _sc_agg_body(x_hbm, src_hbm, dst_hbm, z128_hbm, out_hbm,
                 idxs, idxd, rows, acc, sem):
    cid = lax.axis_index("c")
    sid = lax.axis_index("s")

    sl = pl.ds(sid * N_SLICE, N_SLICE)
    pltpu.sync_copy(z128_hbm.at[sl], acc.at[sl])
    plsc.subcore_barrier()

    def run(nchunks, chunk0):
        def window(w, carry):
            base = chunk0 + w * IDXW
            pltpu.sync_copy(src_hbm.at[pl.ds(base, IDXW)], idxs)
            pltpu.sync_copy(dst_hbm.at[pl.ds(base, IDXW)], idxd)
            cp = pltpu.async_copy(x_hbm.at[idxs.at[0]], rows.at[0], sem)
            for j in range(IDXW):
                cp.wait()
                if j + 1 < IDXW:
                    cp = pltpu.async_copy(
                        x_hbm.at[idxs.at[j + 1]], rows.at[(j + 1) % 2], sem)
                pltpu.sync_copy(rows.at[j % 2], acc.at[idxd.at[j]], add=True)
            return carry

        lax.fori_loop(0, nchunks // IDXW, window, 0)

    @pl.when(cid == 0)
    def _():
        run(C_CORE0, sid * C_CORE0)

    @pl.when(cid == 1)
    def _():
        run(C_CORE1, NS * C_CORE0 + sid * C_CORE1)

    plsc.subcore_barrier()
    pltpu.sync_copy(acc.at[sl], out_hbm.at[cid, sl])


@functools.lru_cache(maxsize=None)
def _get_sc_agg():
    return pl.kernel(
        _sc_agg_body,
        out_type=[jax.ShapeDtypeStruct((NC, N_PAD, D), jnp.float32)],
        mesh=plsc.VectorSubcoreMesh(core_axis_name="c", subcore_axis_name="s"),
        scratch_types=[
            pltpu.VMEM((IDXW, CHUNK), jnp.int32),
            pltpu.VMEM((IDXW, CHUNK), jnp.int32),
            pltpu.VMEM((2, CHUNK, D), jnp.float32),
            pltpu.VMEM_SHARED((N_PAD, D), jnp.float32),
            pltpu.SemaphoreType.DMA,
        ],
    )


def _sc_deg_body(dst_hbm, z128_hbm, ones_hbm, deg_hbm, idxd, ones_v, dacc):
    cid = lax.axis_index("c")
    sid = lax.axis_index("s")
    wid = cid * NS + sid

    pltpu.sync_copy(dst_hbm.at[pl.ds(wid * NCHUNKS, NCHUNKS)], idxd)
    sl = pl.ds(sid * N_SLICE, N_SLICE)
    pltpu.sync_copy(z128_hbm.at[sl], dacc.at[sl])
    pltpu.sync_copy(ones_hbm, ones_v)
    plsc.subcore_barrier()

    def step(c, carry):
        pltpu.sync_copy(ones_v, dacc.at[idxd.at[c]], add=True)
        return carry

    lax.fori_loop(0, NCHUNKS, step, 0)

    plsc.subcore_barrier()
    pltpu.sync_copy(dacc.at[sl], deg_hbm.at[cid, sl])


@functools.lru_cache(maxsize=None)
def _get_sc_deg():
    return pl.kernel(
        _sc_deg_body,
        out_type=[jax.ShapeDtypeStruct((NC, N_PAD, D), jnp.float32)],
        mesh=plsc.VectorSubcoreMesh(core_axis_name="c", subcore_axis_name="s"),
        scratch_types=[
            pltpu.VMEM((NCHUNKS, CHUNK), jnp.int32),
            pltpu.VMEM((CHUNK, D), jnp.float32),
            pltpu.VMEM_SHARED((N_PAD, D), jnp.float32),
        ],
    )


def _tc_layer_body(final, p_ref, dg_ref, x_ref, wl_ref, bl_ref, wr_ref,
                   g_ref, be_ref, *rest):
    if final:
        wm1_ref, bm1_ref, wm2_ref, bm2_ref, o_ref = rest
    else:
        (o_ref,) = rest
    deg = dg_ref[0, :, 0:1] + dg_ref[1, :, 0:1]
    inv = 1.0 / jnp.maximum(deg, 1.0)
    agg = (p_ref[0] + p_ref[1]) * inv
    x = x_ref[...]
    y = (jnp.dot(agg, wl_ref[...], preferred_element_type=jnp.float32)
         + jnp.dot(x, wr_ref[...], preferred_element_type=jnp.float32)
         + bl_ref[...])
    mu = jnp.mean(y, axis=-1, keepdims=True)
    var = jnp.mean((y - mu) * (y - mu), axis=-1, keepdims=True)
    yn = (y - mu) * lax.rsqrt(var + 1e-5) * g_ref[...] + be_ref[...]
    z = jnp.maximum(yn + x, 0.0)
    if final:
        h = jnp.maximum(
            jnp.dot(z, wm1_ref[...], preferred_element_type=jnp.float32)
            + bm1_ref[...], 0.0)
        o_ref[...] = (jnp.dot(h, wm2_ref[...], preferred_element_type=jnp.float32)
                      + bm2_ref[...])
    else:
        o_ref[...] = z


def _tc_layer(part, deg, x, Wl, bl, Wr, g, be, head=None):
    final = head is not None
    full = lambda s: pl.BlockSpec(s, lambda i: (0,) * len(s))
    in_specs = [
        pl.BlockSpec((NC, ROWS_TC, D), lambda i: (0, i, 0)),
        pl.BlockSpec((NC, ROWS_TC, D), lambda i: (0, i, 0)),
        pl.BlockSpec((ROWS_TC, D), lambda i: (i, 0)),
        full((D, D)), full((1, D)), full((D, D)), full((1, D)), full((1, D)),
    ]
    args = [part, deg, x, Wl, bl.reshape(1, D), Wr, g.reshape(1, D),
            be.reshape(1, D)]
    if final:
        Wm1, bm1, Wm2p, bm2p = head
        in_specs += [full((D, D // 2)), full((1, D // 2)),
                     full((D // 2, D)), full((1, D))]
        args += [Wm1, bm1.reshape(1, D // 2), Wm2p, bm2p.reshape(1, D)]
    return pl.pallas_call(
        functools.partial(_tc_layer_body, final),
        grid=(GRID_TC,),
        in_specs=in_specs,
        out_specs=pl.BlockSpec((ROWS_TC, D), lambda i: (i, 0)),
        out_shape=jax.ShapeDtypeStruct((N, D), jnp.float32),
    )(*args)


def kernel(x, edge_index, Wl0, bl0, Wr0, g0, be0, Wl1, bl1, Wr1, g1, be1,
           Wl2, bl2, Wr2, g2, be2, Wm1, bm1, Wm2, bm2):
    pad = E_PAD - E
    src = jnp.concatenate([edge_index[0], jnp.zeros((pad,), jnp.int32)])
    dst = jnp.concatenate([edge_index[1], jnp.full((pad,), N, jnp.int32)])
    src2 = src.reshape(E_PAD // CHUNK, CHUNK)
    dst2 = dst.reshape(E_PAD // CHUNK, CHUNK)
    z128 = jnp.zeros((N_PAD, D), jnp.float32)
    ones = jnp.ones((CHUNK, D), jnp.float32)
    Wm2p = jnp.pad(Wm2, ((0, 0), (0, D - Wm2.shape[1])))
    bm2p = jnp.pad(bm2, (0, D - bm2.shape[0]))

    sc_deg, sc_agg = _get_sc_deg(), _get_sc_agg()
    (deg,) = sc_deg(dst2, z128, ones)
    (part,) = sc_agg(x, src2, dst2, z128)
    x1 = _tc_layer(part, deg, x, Wl0, bl0, Wr0, g0, be0)
    (part,) = sc_agg(x1, src2, dst2, z128)
    x2 = _tc_layer(part, deg, x1, Wl1, bl1, Wr1, g1, be1)
    (part,) = sc_agg(x2, src2, dst2, z128)
    out = _tc_layer(part, deg, x2, Wl2, bl2, Wr2, g2, be2,
                    head=(Wm1, bm1, Wm2p, bm2p))
    return out[:, :Wm2.shape[1]]


# speedup vs baseline: 1.0962x; 1.0962x over previous
import functools

import jax
import jax.numpy as jnp
from jax import lax
from jax.experimental import pallas as pl
from jax.experimental.pallas import tpu as pltpu
from jax.experimental.pallas import tpu_sc as plsc

N = 10000
D = 128
E = 320000

NC = 2
NS = 16
NW = NC * NS
CHUNK = 128
NCHUNKS = 80
E_PER_TILE = CHUNK * NCHUNKS
E_PAD = E_PER_TILE * NW
N_SLICE = 632
N_PAD = N_SLICE * NS
ROWS_TC = 400
GRID_TC = N // ROWS_TC


C_CORE0 = 144
C_CORE1 = 2 * NCHUNKS - C_CORE0
IDXW = 8


def _sc_agg_body(x_hbm, src_hbm, dst_hbm, z128_hbm, out_hbm,
                 idxs, idxd, rows, acc, sem):
    cid = lax.axis_index("c")
    sid = lax.axis_index("s")

    sl = pl.ds(sid * N_SLICE, N_SLICE)
    pltpu.sync_copy(z128_hbm.at[sl], acc.at[sl])
    plsc.subcore_barrier()

    def run(nchunks, chunk0):
        def window(w, carry):
            base = chunk0 + w * IDXW
            pltpu.sync_copy(src_hbm.at[pl.ds(base, IDXW)], idxs)
            pltpu.sync_copy(dst_hbm.at[pl.ds(base, IDXW)], idxd)
            cp = pltpu.async_copy(x_hbm.at[idxs.at[0]], rows.at[0], sem)
            for j in range(IDXW):
                cp.wait()
                if j + 1 < IDXW:
                    cp = pltpu.async_copy(
                        x_hbm.at[idxs.at[j + 1]], rows.at[(j + 1) % 2], sem)
                pltpu.sync_copy(rows.at[j % 2], acc.at[idxd.at[j]], add=True)
            return carry

        lax.fori_loop(0, nchunks // IDXW, window, 0)

    @pl.when(cid == 0)
    def _():
        run(C_CORE0, sid * C_CORE0)

    @pl.when(cid == 1)
    def _():
        run(C_CORE1, NS * C_CORE0 + sid * C_CORE1)

    plsc.subcore_barrier()
    pltpu.sync_copy(acc.at[sl], out_hbm.at[cid, sl])


@functools.lru_cache(maxsize=None)
def _get_sc_agg():
    return pl.kernel(
        _sc_agg_body,
        out_type=[jax.ShapeDtypeStruct((NC, N_PAD, D), jnp.float32)],
        mesh=plsc.VectorSubcoreMesh(core_axis_name="c", subcore_axis_name="s"),
        scratch_types=[
            pltpu.VMEM((IDXW, CHUNK), jnp.int32),
            pltpu.VMEM((IDXW, CHUNK), jnp.int32),
            pltpu.VMEM((2, CHUNK, D), jnp.float32),
            pltpu.VMEM_SHARED((N_PAD, D), jnp.float32),
            pltpu.SemaphoreType.DMA,
        ],
    )


def _sc_deg_body(dst_hbm, z128_hbm, ones_hbm, deg_hbm, idxd, ones_v, dacc):
    cid = lax.axis_index("c")
    sid = lax.axis_index("s")
    wid = cid * NS + sid

    pltpu.sync_copy(dst_hbm.at[pl.ds(wid * NCHUNKS, NCHUNKS)], idxd)
    sl = pl.ds(sid * N_SLICE, N_SLICE)
    pltpu.sync_copy(z128_hbm.at[sl], dacc.at[sl])
    pltpu.sync_copy(ones_hbm, ones_v)
    plsc.subcore_barrier()

    def step(c, carry):
        pltpu.sync_copy(ones_v, dacc.at[idxd.at[c]], add=True)
        return carry

    lax.fori_loop(0, NCHUNKS, step, 0)

    plsc.subcore_barrier()
    pltpu.sync_copy(dacc.at[sl], deg_hbm.at[cid, sl])


@functools.lru_cache(maxsize=None)
def _get_sc_deg():
    return pl.kernel(
        _sc_deg_body,
        out_type=[jax.ShapeDtypeStruct((NC, N_PAD, D), jnp.float32)],
        mesh=plsc.VectorSubcoreMesh(core_axis_name="c", subcore_axis_name="s"),
        scratch_types=[
            pltpu.VMEM((NCHUNKS, CHUNK), jnp.int32),
            pltpu.VMEM((CHUNK, D), jnp.float32),
            pltpu.VMEM_SHARED((N_PAD, D), jnp.float32),
        ],
    )


def _tc_layer_body(final, p_ref, dg_ref, x_ref, wl_ref, bl_ref, wr_ref,
                   g_ref, be_ref, *rest):
    if final:
        wm1_ref, bm1_ref, wm2_ref, bm2_ref, o_ref = rest
    else:
        (o_ref,) = rest
    deg = dg_ref[0, :, 0:1] + dg_ref[1, :, 0:1]
    inv = 1.0 / jnp.maximum(deg, 1.0)
    agg = (p_ref[0] + p_ref[1]) * inv
    x = x_ref[...]
    y = (jnp.dot(agg, wl_ref[...], preferred_element_type=jnp.float32)
         + jnp.dot(x, wr_ref[...], preferred_element_type=jnp.float32)
         + bl_ref[...])
    mu = jnp.mean(y, axis=-1, keepdims=True)
    var = jnp.mean((y - mu) * (y - mu), axis=-1, keepdims=True)
    yn = (y - mu) * lax.rsqrt(var + 1e-5) * g_ref[...] + be_ref[...]
    z = jnp.maximum(yn + x, 0.0)
    if final:
        h = jnp.maximum(
            jnp.dot(z, wm1_ref[...], preferred_element_type=jnp.float32)
            + bm1_ref[...], 0.0)
        o_ref[...] = (jnp.dot(h, wm2_ref[...], preferred_element_type=jnp.float32)
                      + bm2_ref[...])
    else:
        o_ref[...] = z


def _tc_layer(part, deg, x, Wl, bl, Wr, g, be, head=None):
    final = head is not None
    full = lambda s: pl.BlockSpec(s, lambda i: (0,) * len(s))
    in_specs = [
        pl.BlockSpec((NC, ROWS_TC, D), lambda i: (0, i, 0)),
        pl.BlockSpec((NC, ROWS_TC, D), lambda i: (0, i, 0)),
        pl.BlockSpec((ROWS_TC, D), lambda i: (i, 0)),
        full((D, D)), full((1, D)), full((D, D)), full((1, D)), full((1, D)),
    ]
    args = [part, deg, x, Wl, bl.reshape(1, D), Wr, g.reshape(1, D),
            be.reshape(1, D)]
    if final:
        Wm1, bm1, Wm2p, bm2p = head
        in_specs += [full((D, D // 2)), full((1, D // 2)),
                     full((D // 2, D)), full((1, D))]
        args += [Wm1, bm1.reshape(1, D // 2), Wm2p, bm2p.reshape(1, D)]
    return pl.pallas_call(
        functools.partial(_tc_layer_body, final),
        grid=(GRID_TC,),
        in_specs=in_specs,
        out_specs=pl.BlockSpec((ROWS_TC, D), lambda i: (i, 0)),
        out_shape=jax.ShapeDtypeStruct((N, D), jnp.float32),
    )(*args)


def kernel(x, edge_index, Wl0, bl0, Wr0, g0, be0, Wl1, bl1, Wr1, g1, be1,
           Wl2, bl2, Wr2, g2, be2, Wm1, bm1, Wm2, bm2):
    pad = E_PAD - E
    src = jnp.concatenate([edge_index[0], jnp.zeros((pad,), jnp.int32)])
    dst = jnp.concatenate([edge_index[1], jnp.full((pad,), N, jnp.int32)])
    src2 = src.reshape(E_PAD // CHUNK, CHUNK)
    dst2 = dst.reshape(E_PAD // CHUNK, CHUNK)
    z128 = jnp.zeros((N_PAD, D), jnp.float32)
    ones = jnp.ones((CHUNK, D), jnp.float32)
    Wm2p = jnp.pad(Wm2, ((0, 0), (0, D - Wm2.shape[1])))
    bm2p = jnp.pad(bm2, (0, D - bm2.shape[0]))

    sc_deg, sc_agg = _get_sc_deg(), _get_sc_agg()
    (deg,) = sc_deg(dst2, z128, ones)
    (part,) = sc_agg(x, src2, dst2, z128)
    x1 = _tc_layer(part, deg, x, Wl0, bl0, Wr0, g0, be0)
    (part,) = sc_agg(x1, src2, dst2, z128)
    x2 = _tc_layer(part, deg, x1, Wl1, bl1, Wr1, g1, be1)
    (part,) = sc_agg(x2, src2, dst2, z128)
    out = _tc_layer(part, deg, x2, Wl2, bl2, Wr2, g2, be2,
                    head=(Wm1, bm1, Wm2p, bm2p))
    return out[:, :Wm2.shape[1]]
